# baseline (speedup 1.0000x reference)
import functools

import jax
import jax.numpy as jnp
from jax import lax
from jax.experimental import pallas as pl
from jax.experimental.pallas import tpu as pltpu
from jax.experimental.pallas import tpu_sc as plsc

_N_TIME = 4096
_EMBED_DIM = 2048
_OUT_WIDTH = 2048
_BATCH = 4
_TOTAL = _BATCH * _N_TIME

_NUM_CORES = 2
_NUM_SUBCORES = 16
_NW = _NUM_CORES * _NUM_SUBCORES
_ROWS_PER_W = _TOTAL // _NW
_CHUNK = 16
_NCHUNK = _ROWS_PER_W // _CHUNK
_W = 24
_WMAX = _EMBED_DIM - _W


@functools.partial(
    pl.kernel,
    out_type=jax.ShapeDtypeStruct((_TOTAL, _OUT_WIDTH), jnp.float32),
    mesh=plsc.VectorSubcoreMesh(core_axis_name="c", subcore_axis_name="s"),
    scratch_types=[
        pltpu.VMEM((2, 16), jnp.float32),
        pltpu.VMEM((_W, _OUT_WIDTH), jnp.float32),
        pltpu.VMEM((_W, _OUT_WIDTH), jnp.float32),
        pltpu.SemaphoreType.DMA,
        pltpu.SemaphoreType.DMA,
        pltpu.SemaphoreType.DMA,
        pltpu.SemaphoreType.DMA,
    ],
)
def _range_embed(params_hbm, emb_hbm, out_hbm,
                 params_v, win0, win1, gsem0, gsem1, ssem0, ssem1):
    wid = lax.axis_index("s") * _NUM_CORES + lax.axis_index("c")
    base = wid * _ROWS_PER_W
    b = base // _N_TIME
    t0 = base - b * _N_TIME

    pltpu.sync_copy(params_hbm.at[wid], params_v)
    ps = params_v[0, :]
    pe = params_v[1, :]
    delta = pe - ps
    lanes = lax.iota(jnp.int32, 16)

    wins = (win0, win1)
    gsems = (gsem0, gsem1)
    ssems = (ssem0, ssem1)

    def bins_of(g):
        t = lanes + (t0 + g * _CHUNK)
        interp = t.astype(jnp.float32) * (1.0 / _N_TIME)
        pos = ps + delta * interp
        bins = (jnp.float32(_EMBED_DIM) * pos).astype(jnp.int32)
        return jnp.minimum(jnp.maximum(bins, 0), _EMBED_DIM - 1)

    def span_of(bins):
        a = bins[0]
        c = bins[15]
        return jnp.minimum(a, c), jnp.maximum(a, c), c >= a

    def win_base_for(m, mx, inc):
        wb = jnp.where(inc, (m // 8) * 8, ((mx - 16) // 8) * 8)
        return jnp.minimum(jnp.maximum(wb, 0), _WMAX)

    def fetch(s, w):
        wb = pl.multiple_of(w, 8)
        pltpu.async_copy(emb_hbm.at[pl.ds(wb, _W)], wins[s], gsems[s])

    def fetch_wait(s):
        pltpu.make_async_copy(
            emb_hbm.at[pl.ds(0, _W)], wins[s], gsems[s]
        ).wait()

    def drain_chunk(g, gmod2):
        row0 = base + g * _CHUNK
        sem = ssems[gmod2]
        for r in range(_CHUNK):
            pltpu.make_async_copy(
                wins[0].at[pl.ds(0, 1)],
                out_hbm.at[pl.ds(row0 + r, 1)],
                sem,
            ).wait()

    def chunk_body(g, gmod2, s, w, fresh):
        pl.when(g >= 2)(lambda: drain_chunk(g - 2, gmod2))
        pl.when(fresh & (s == 0))(lambda: fetch_wait(0))
        pl.when(fresh & (s == 1))(lambda: fetch_wait(1))
        bins = bins_of(g)

        def scat(slot):
            row0 = base + g * _CHUNK
            sem = ssems[gmod2]
            for r in range(_CHUNK):
                off_r = bins[r] - w
                pltpu.async_copy(
                    wins[slot].at[pl.ds(off_r, 1)],
                    out_hbm.at[pl.ds(row0 + r, 1)],
                    sem,
                )

        pl.when(s == 0)(lambda: scat(0))
        pl.when(s == 1)(lambda: scat(1))
        m, mx, inc = span_of(bins_of(g + 1))
        cov = ((m >= w) & (mx < w + _W)) | (g == _NCHUNK - 1)
        wn = win_base_for(m, mx, inc)
        nf = jnp.logical_not(cov)
        pl.when(nf & (s == 0))(lambda: fetch(1, wn))
        pl.when(nf & (s == 1))(lambda: fetch(0, wn))
        s_out = jnp.where(cov, s, 1 - s)
        w_out = jnp.where(cov, w, wn)
        return s_out, w_out, nf

    m0, mx0, inc0 = span_of(bins_of(0))
    w0 = win_base_for(m0, mx0, inc0)
    fetch(0, w0)

    def pair_loop(i, carry):
        s, w, fresh = carry
        for j in (0, 1):
            g = 2 * i + j
            s, w, fresh = chunk_body(g, j, s, w, fresh)
        return s, w, fresh

    lax.fori_loop(
        0, _NCHUNK // 2, pair_loop, (jnp.int32(0), w0, jnp.bool_(True))
    )
    drain_chunk(_NCHUNK - 2, (_NCHUNK - 2) % 2)
    drain_chunk(_NCHUNK - 1, (_NCHUNK - 1) % 2)


def kernel(pos_start, pos_end, emb):
    ps = pos_start.astype(jnp.float32).reshape(-1)
    pe = pos_end.astype(jnp.float32).reshape(-1)
    reps = _NW // _BATCH
    ps_w = jnp.repeat(ps, reps)
    pe_w = jnp.repeat(pe, reps)
    params = jnp.stack([ps_w, pe_w], axis=1)
    params = jnp.broadcast_to(params[:, :, None], (_NW, 2, 16))
    out = _range_embed(params, emb)
    return out.reshape(_BATCH, _N_TIME, _OUT_WIDTH)
